# baseline (speedup 1.0000x reference)
import functools

import jax
import jax.numpy as jnp
from jax import lax
from jax.experimental import pallas as pl
from jax.experimental.pallas import tpu as pltpu
from jax.experimental.pallas import tpu_sc as plsc

NUM_FIELDS = 26
VOCAB = 100000
EMBED_DIM = 16
BATCH = 16384

N_ROWS = BATCH * NUM_FIELDS
NW = 32
ROWS_PER_W = N_ROWS // NW
CHUNK = 3328
N_CHUNKS = ROWS_PER_W // CHUNK
VECS_PER_CHUNK = CHUNK // 16


def _build_sc_gather():
    mesh = plsc.VectorSubcoreMesh(core_axis_name="c", subcore_axis_name="s")

    @functools.partial(
        pl.kernel,
        out_type=jax.ShapeDtypeStruct((N_ROWS, EMBED_DIM), jnp.float32),
        mesh=mesh,
        scratch_types=[
            pltpu.VMEM((CHUNK,), jnp.int32),
            pltpu.VMEM((CHUNK,), jnp.int32),
            pltpu.VMEM((CHUNK, EMBED_DIM), jnp.float32),
            pltpu.SemaphoreType.DMA,
        ],
        compiler_params=pltpu.CompilerParams(use_tc_tiling_on_sc=False),
    )
    def gather_kernel(x_hbm, offs_hbm, tab_hbm, out_hbm,
                      idx_v, offs_v, rows_v, sem):
        wid = lax.axis_index("s") * 2 + lax.axis_index("c")
        base = wid * ROWS_PER_W
        pltpu.sync_copy(offs_hbm, offs_v)

        def do_chunk(c, carry):
            start = base + c * CHUNK
            pltpu.sync_copy(x_hbm.at[pl.ds(start, CHUNK)], idx_v)

            def add_offs(p, carry2):
                sl = pl.ds(p * 16, 16)
                idx_v[sl] = idx_v[sl] + offs_v[sl]
                return carry2

            lax.fori_loop(0, VECS_PER_CHUNK, add_offs, 0)
            pltpu.async_copy(tab_hbm.at[idx_v], rows_v, sem).wait()
            pltpu.sync_copy(rows_v, out_hbm.at[pl.ds(start, CHUNK)])
            return carry

        lax.fori_loop(0, N_CHUNKS, do_chunk, 0)

    return gather_kernel


_sc_gather = _build_sc_gather()


@jax.jit
def kernel(x, tables):
    x_flat = x.reshape(-1).astype(jnp.int32)
    tab_flat = tables.reshape(NUM_FIELDS * VOCAB, EMBED_DIM)
    offs = (jnp.arange(CHUNK, dtype=jnp.int32) % NUM_FIELDS) * VOCAB
    out = _sc_gather(x_flat, offs, tab_flat)
    return out.reshape(BATCH, NUM_FIELDS, EMBED_DIM)


# speedup vs baseline: 1.0015x; 1.0015x over previous
import functools

import jax
import jax.numpy as jnp
from jax import lax
from jax.experimental import pallas as pl
from jax.experimental.pallas import tpu as pltpu
from jax.experimental.pallas import tpu_sc as plsc

NUM_FIELDS = 26
VOCAB = 100000
EMBED_DIM = 16
BATCH = 16384

N_ROWS = BATCH * NUM_FIELDS
NW = 32
ROWS_PER_W = N_ROWS // NW
CHUNK = 3328
N_CHUNKS = ROWS_PER_W // CHUNK
VECS_PER_CHUNK = CHUNK // 16


def _build_sc_gather():
    mesh = plsc.VectorSubcoreMesh(core_axis_name="c", subcore_axis_name="s")

    @functools.partial(
        pl.kernel,
        out_type=jax.ShapeDtypeStruct((N_ROWS, EMBED_DIM), jnp.float32),
        mesh=mesh,
        scratch_types=[
            pltpu.VMEM((CHUNK,), jnp.int32),
            pltpu.VMEM((CHUNK,), jnp.int32),
            pltpu.VMEM((CHUNK, EMBED_DIM), jnp.float32),
            pltpu.SemaphoreType.DMA,
        ],
        compiler_params=pltpu.CompilerParams(use_tc_tiling_on_sc=False),
    )
    def gather_kernel(gidx_hbm, tab_hbm, out_hbm,
                      idx_v, offs_v, rows_v, sem):
        wid = lax.axis_index("s") * 2 + lax.axis_index("c")
        base = wid * ROWS_PER_W

        def do_chunk(c, carry):
            start = base + c * CHUNK
            pltpu.sync_copy(gidx_hbm.at[pl.ds(start, CHUNK)], idx_v)
            pltpu.async_copy(tab_hbm.at[idx_v], rows_v, sem).wait()
            pltpu.sync_copy(rows_v, out_hbm.at[pl.ds(start, CHUNK)])
            return carry

        lax.fori_loop(0, N_CHUNKS, do_chunk, 0)

    return gather_kernel


_sc_gather = _build_sc_gather()


@jax.jit
def kernel(x, tables):
    gidx = (x.astype(jnp.int32)
            + jnp.arange(NUM_FIELDS, dtype=jnp.int32)[None, :] * VOCAB)
    gidx = gidx.reshape(-1)
    tab_flat = tables.reshape(NUM_FIELDS * VOCAB, EMBED_DIM)
    out = _sc_gather(gidx, tab_flat)
    return out.reshape(BATCH, NUM_FIELDS, EMBED_DIM)


# speedup vs baseline: 2.5142x; 2.5105x over previous
import functools

import jax
import jax.numpy as jnp
from jax import lax
from jax.experimental import pallas as pl
from jax.experimental.pallas import tpu as pltpu
from jax.experimental.pallas import tpu_sc as plsc

NUM_FIELDS = 26
VOCAB = 100000
EMBED_DIM = 16
BATCH = 16384

N_JOBS = NUM_FIELDS * EMBED_DIM
NW = 32
JOBS_PER_W = N_JOBS // NW
VECS = BATCH // 16


def _build_sc_gather():
    mesh = plsc.VectorSubcoreMesh(core_axis_name="c", subcore_axis_name="s")

    @functools.partial(
        pl.kernel,
        out_type=jax.ShapeDtypeStruct((N_JOBS, BATCH), jnp.float32),
        mesh=mesh,
        scratch_types=[
            pltpu.VMEM((BATCH,), jnp.int32),
            pltpu.VMEM((BATCH,), jnp.int32),
            pltpu.VMEM((2, BATCH), jnp.float32),
            pltpu.SemaphoreType.DMA,
            pltpu.SemaphoreType.DMA,
        ],
        compiler_params=pltpu.CompilerParams(use_tc_tiling_on_sc=False),
    )
    def gather_kernel(xt_hbm, tab_hbm, out_hbm, col_v, idx_v, rows_v,
                      g_sem, w_sem):
        wid = lax.axis_index("s") * 2 + lax.axis_index("c")
        j0 = wid * JOBS_PER_W

        def do_job(t, carry):
            j = j0 + t
            f = j // EMBED_DIM
            pltpu.sync_copy(xt_hbm.at[f], col_v)
            shift = j * VOCAB

            def add_shift(p, c2):
                sl = pl.ds(p * 16, 16)
                idx_v[sl] = col_v[sl] + shift
                return c2

            lax.fori_loop(0, VECS, add_shift, 0)
            buf = lax.rem(t, 2)
            pltpu.async_copy(tab_hbm.at[idx_v], rows_v.at[buf], g_sem).wait()
            pltpu.sync_copy(rows_v.at[buf], out_hbm.at[j])
            return carry

        lax.fori_loop(0, JOBS_PER_W, do_job, 0)

    return gather_kernel


_sc_gather = _build_sc_gather()


@jax.jit
def kernel(x, tables):
    xt = x.astype(jnp.int32).T
    tab_flat = tables.transpose(0, 2, 1).reshape(-1)
    out = _sc_gather(xt, tab_flat)
    return out.reshape(NUM_FIELDS, EMBED_DIM, BATCH).transpose(2, 0, 1)


# speedup vs baseline: 2.8415x; 1.1302x over previous
import functools

import jax
import jax.numpy as jnp
from jax import lax
from jax.experimental import pallas as pl
from jax.experimental.pallas import tpu as pltpu
from jax.experimental.pallas import tpu_sc as plsc

NUM_FIELDS = 26
VOCAB = 100000
EMBED_DIM = 16
BATCH = 16384

N_JOBS = NUM_FIELDS * EMBED_DIM
NW = 32
JOBS_PER_W = N_JOBS // NW
VECS = BATCH // 16
UNROLL = 8


def _build_sc_gather():
    mesh = plsc.VectorSubcoreMesh(core_axis_name="c", subcore_axis_name="s")

    @functools.partial(
        pl.kernel,
        out_type=jax.ShapeDtypeStruct((N_JOBS, BATCH), jnp.float32),
        mesh=mesh,
        scratch_types=[
            pltpu.VMEM((BATCH,), jnp.int32),
            pltpu.VMEM((2, BATCH), jnp.int32),
            pltpu.VMEM((2, BATCH), jnp.float32),
            pltpu.SemaphoreType.DMA,
            pltpu.SemaphoreType.DMA,
        ],
        compiler_params=pltpu.CompilerParams(use_tc_tiling_on_sc=False),
    )
    def gather_kernel(xt_hbm, tab_hbm, out_hbm, col_v, idx_v, rows_v,
                      g_sem, w_sem):
        wid = lax.axis_index("s") * 2 + lax.axis_index("c")
        j0 = wid * JOBS_PER_W

        def prep(t):
            j = j0 + t
            f = j // EMBED_DIM

            @pl.when(jnp.logical_or(t == 0, f * EMBED_DIM == j))
            def _():
                pltpu.sync_copy(xt_hbm.at[f], col_v)

            shift = j * VOCAB
            buf = lax.rem(t, 2)

            def add_shift(p, c2):
                base = p * (16 * UNROLL)
                for u in range(UNROLL):
                    sl = pl.ds(base + u * 16, 16)
                    idx_v[buf, sl] = col_v[sl] + shift
                return c2

            lax.fori_loop(0, VECS // UNROLL, add_shift, 0)

        prep(0)

        def do_job(t, carry):
            buf = lax.rem(t, 2)

            @pl.when(t >= 2)
            def _():
                pltpu.make_async_copy(rows_v.at[buf], out_hbm.at[0],
                                      w_sem).wait()

            gather = pltpu.async_copy(tab_hbm.at[idx_v.at[buf]],
                                      rows_v.at[buf], g_sem)

            @pl.when(t + 1 < JOBS_PER_W)
            def _():
                prep(t + 1)

            gather.wait()
            pltpu.async_copy(rows_v.at[buf], out_hbm.at[j0 + t], w_sem)
            return carry

        lax.fori_loop(0, JOBS_PER_W, do_job, 0)
        pltpu.make_async_copy(rows_v.at[lax.rem(JOBS_PER_W, 2)],
                              out_hbm.at[0], w_sem).wait()
        pltpu.make_async_copy(rows_v.at[lax.rem(JOBS_PER_W + 1, 2)],
                              out_hbm.at[0], w_sem).wait()

    return gather_kernel


_sc_gather = _build_sc_gather()


@jax.jit
def kernel(x, tables):
    xt = x.astype(jnp.int32).T
    tab_flat = tables.transpose(0, 2, 1).reshape(-1)
    out = _sc_gather(xt, tab_flat)
    return out.reshape(NUM_FIELDS, EMBED_DIM, BATCH).transpose(2, 0, 1)


# speedup vs baseline: 7.2710x; 2.5589x over previous
import functools

import jax
import jax.numpy as jnp
from jax import lax
from jax.experimental import pallas as pl
from jax.experimental.pallas import tpu as pltpu
from jax.experimental.pallas import tpu_sc as plsc

NUM_FIELDS = 26
VOCAB = 100000
EMBED_DIM = 16
BATCH = 16384

N_JOBS = NUM_FIELDS * EMBED_DIM
NW = 32
JOBS_PER_W = N_JOBS // NW
CHUNK = 4096
N_CHUNKS = BATCH // CHUNK
UNROLL = 8


def _build_sc_gather():
    mesh = plsc.VectorSubcoreMesh(core_axis_name="c", subcore_axis_name="s")

    @functools.partial(
        pl.kernel,
        out_type=jax.ShapeDtypeStruct((N_JOBS, BATCH), jnp.float32),
        mesh=mesh,
        scratch_types=[
            pltpu.VMEM((VOCAB,), jnp.float32),
            pltpu.VMEM((BATCH,), jnp.int32),
            pltpu.VMEM((2, CHUNK), jnp.float32),
            pltpu.SemaphoreType.DMA,
        ],
        compiler_params=pltpu.CompilerParams(use_tc_tiling_on_sc=True,
                                             needs_layout_passes=False),
    )
    def gather_kernel(xt_hbm, tab_hbm, out_hbm, slab_v, col_v, obuf_v,
                      w_sem):
        wid = lax.axis_index("s") * 2 + lax.axis_index("c")
        j0 = wid * JOBS_PER_W

        def do_job(t, carry):
            j = j0 + t
            f = j // EMBED_DIM
            pltpu.sync_copy(tab_hbm.at[f, j - f * EMBED_DIM], slab_v)

            @pl.when(jnp.logical_or(t == 0, f * EMBED_DIM == j))
            def _():
                pltpu.sync_copy(xt_hbm.at[f], col_v)

            for k in range(N_CHUNKS):
                half = k % 2
                if k >= 2:
                    pltpu.make_async_copy(
                        obuf_v.at[half],
                        out_hbm.at[j0, pl.ds(0, CHUNK)], w_sem).wait()
                elif k < 2:
                    @pl.when(t > 0)
                    def _():
                        pltpu.make_async_copy(
                            obuf_v.at[half],
                            out_hbm.at[j0, pl.ds(0, CHUNK)], w_sem).wait()

                def gath(p, c2):
                    base = k * CHUNK + p * (16 * UNROLL)
                    for u in range(UNROLL):
                        sl = pl.ds(base + u * 16, 16)
                        osl = pl.ds(base + u * 16 - k * CHUNK, 16)
                        idx = col_v[sl]
                        obuf_v[half, osl] = plsc.load_gather(slab_v, [idx])
                    return c2

                lax.fori_loop(0, CHUNK // (16 * UNROLL), gath, 0)
                pltpu.async_copy(obuf_v.at[half],
                                 out_hbm.at[j, pl.ds(k * CHUNK, CHUNK)],
                                 w_sem)
            return carry

        lax.fori_loop(0, JOBS_PER_W, do_job, 0)
        for _ in range(2):
            pltpu.make_async_copy(obuf_v.at[0],
                                  out_hbm.at[j0, pl.ds(0, CHUNK)],
                                  w_sem).wait()

    return gather_kernel


_sc_gather = _build_sc_gather()


@jax.jit
def kernel(x, tables):
    xt = x.astype(jnp.int32).T
    tabt = tables.transpose(0, 2, 1)
    out = _sc_gather(xt, tabt)
    return out.reshape(NUM_FIELDS, EMBED_DIM, BATCH).transpose(2, 0, 1)
